# Initial kernel scaffold; baseline (speedup 1.0000x reference)
import jax
import jax.numpy as jnp
from jax.experimental import pallas as pl


def kernel(verts_attr, bary_coords, vt_to_v_index, faces_uv, pix_to_face):
    raise NotImplementedError("write your pallas kernel here")


# speedup vs baseline: 3.7322x; 3.7322x over previous
import dataclasses
import functools

import jax
import jax.numpy as jnp
from jax import lax
from jax.experimental import pallas as pl
from jax.experimental.pallas import tpu as pltpu
from jax.experimental.pallas import tpu_sc as plsc

_NC, _NS, _L = 2, 16, 16
_NW = _NC * _NS


def _uv_render_sc(verts_flat, bary_flat, vt_to_v, fuv_flat, pf_flat, *, B, V, D, F, P):
    C = 32
    PW = P // _NW
    NCHUNK = PW // C
    mesh = plsc.VectorSubcoreMesh(core_axis_name="c", subcore_axis_name="s")
    cp = pltpu.CompilerParams()
    if "needs_layout_passes" in pltpu.CompilerParams.__dataclass_fields__:
        cp = dataclasses.replace(cp, needs_layout_passes=False)
    if "use_tc_tiling_on_sc" in pltpu.CompilerParams.__dataclass_fields__:
        cp = dataclasses.replace(cp, use_tc_tiling_on_sc=False)

    @functools.partial(
        pl.kernel,
        compiler_params=cp,
        out_type=jax.ShapeDtypeStruct((B, P, D), jnp.float32),
        mesh=mesh,
        scratch_types=[
            pltpu.VMEM((vt_to_v.shape[0],), jnp.int32),
            pltpu.VMEM((fuv_flat.shape[0],), jnp.int32),
            pltpu.VMEM((C,), jnp.int32),
            pltpu.VMEM((16 + 3 * C,), jnp.float32),
            pltpu.VMEM((B, 3 * C), jnp.int32),
            pltpu.VMEM((B, 3 * C, D), jnp.float32),
            pltpu.VMEM((B, C, D), jnp.float32),
            pltpu.SemaphoreType.DMA,
        ],
    )
    def k(verts_hbm, bary_hbm, vt_hbm, fuv_hbm, pf_hbm, out_hbm,
          vt_v, fuv_v, pf_v, bary_v, idx_v, gbuf, obuf, sem):
        wid = lax.axis_index("s") * _NC + lax.axis_index("c")
        pltpu.sync_copy(vt_hbm, vt_v)
        pltpu.sync_copy(fuv_hbm, fuv_v)

        @pl.loop(0, NCHUNK)
        def _(c):
            base = wid * PW + c * C
            pltpu.sync_copy(pf_hbm.at[pl.ds(base, C)], pf_v)
            pltpu.sync_copy(bary_hbm.at[pl.ds(3 * base, 3 * C)],
                            bary_v.at[pl.ds(16, 3 * C)])
            for g in range(C // _L):
                pf = pf_v[pl.ds(g * _L, _L)]
                pf = jnp.minimum(jnp.maximum(pf, 0), F - 1)
                for kk in range(3):
                    uv = plsc.load_gather(fuv_v, [pf * 3 + kk])
                    vi = plsc.load_gather(vt_v, [uv])
                    for b in range(B):
                        idx_v[b, pl.ds(kk * C + g * _L, _L)] = vi + b * V
            copies = [
                pltpu.async_copy(verts_hbm.at[idx_v.at[b]], gbuf.at[b], sem)
                for b in range(B)
            ]
            for cp in copies:
                cp.wait()
            for p in range(C):
                w = [plsc.load_gather(bary_v, [jnp.full((_L,), 16 + 3 * p + kk, jnp.int32)])
                     for kk in range(3)]
                for b in range(B):
                    for h in range(D // _L):
                        sl = pl.ds(h * _L, _L)
                        acc = (w[0] * gbuf[b, 0 * C + p, sl]
                               + w[1] * gbuf[b, 1 * C + p, sl]
                               + w[2] * gbuf[b, 2 * C + p, sl])
                        obuf[b, p, sl] = acc
            for b in range(B):
                pltpu.sync_copy(obuf.at[b], out_hbm.at[b, pl.ds(base, C)])

    return k(verts_flat, bary_flat, vt_to_v, fuv_flat, pf_flat)


def kernel(verts_attr, bary_coords, vt_to_v_index, faces_uv, pix_to_face):
    B, V, D = verts_attr.shape
    F = faces_uv.shape[0]
    H, W = pix_to_face.shape
    P = H * W
    vt = vt_to_v_index.astype(jnp.int32)
    vt_pad = (-vt.shape[0]) % 16
    if vt_pad:
        vt = jnp.pad(vt, (0, vt_pad))
    out = _uv_render_sc(
        verts_attr.reshape(B * V, D),
        bary_coords.astype(jnp.float32).reshape(P * 3),
        vt,
        faces_uv.astype(jnp.int32).reshape(F * 3),
        pix_to_face.astype(jnp.int32).reshape(P),
        B=B, V=V, D=D, F=F, P=P,
    )
    return out.reshape(B, H, W, D)


# speedup vs baseline: 4.2572x; 1.1407x over previous
import dataclasses
import functools

import jax
import jax.numpy as jnp
from jax import lax
from jax.experimental import pallas as pl
from jax.experimental.pallas import tpu as pltpu
from jax.experimental.pallas import tpu_sc as plsc

_NC, _NS, _L = 2, 16, 16
_NW = _NC * _NS


def _uv_render_sc(verts_flat, bary_flat, vt_to_v, fuv_flat, pf_flat, *, B, V, D, F, P):
    C = 32
    PW = P // _NW
    NCHUNK = PW // C
    NVTP = vt_to_v.shape[0]
    NF3 = fuv_flat.shape[0]
    mesh = plsc.VectorSubcoreMesh(core_axis_name="c", subcore_axis_name="s")
    cp = pltpu.CompilerParams()
    if "needs_layout_passes" in pltpu.CompilerParams.__dataclass_fields__:
        cp = dataclasses.replace(cp, needs_layout_passes=False)
    if "use_tc_tiling_on_sc" in pltpu.CompilerParams.__dataclass_fields__:
        cp = dataclasses.replace(cp, use_tc_tiling_on_sc=False)

    @functools.partial(
        pl.kernel,
        compiler_params=cp,
        out_type=jax.ShapeDtypeStruct((B, P, D), jnp.float32),
        mesh=mesh,
        scratch_types=[
            pltpu.VMEM((NVTP,), jnp.int32),
            pltpu.VMEM((NF3,), jnp.int32),
            pltpu.VMEM((PW,), jnp.int32),
            pltpu.VMEM((16 + 3 * PW,), jnp.float32),
            pltpu.VMEM((2 * B, 3 * C), jnp.int32),
            pltpu.VMEM((2 * B * 3 * C, D), jnp.float32),
            pltpu.VMEM((2 * B * C, D), jnp.float32),
            pltpu.SemaphoreType.DMA,
            pltpu.SemaphoreType.DMA,
            pltpu.SemaphoreType.DMA,
            pltpu.SemaphoreType.DMA,
            pltpu.SemaphoreType.DMA,
        ],
    )
    def k(verts_hbm, bary_hbm, vt_hbm, fuv_hbm, pf_hbm, out_hbm,
          vt_v, fuv_v, pf_all, bary_all, idx_v, gbuf, obuf,
          sem_in, sem_g0, sem_g1, sem_o0, sem_o1):
        wid = lax.axis_index("s") * _NC + lax.axis_index("c")
        iota = lax.iota(jnp.int32, _L)

        def splat(v):
            return jnp.full((_L,), v, jnp.int32)

        ins = [
            pltpu.async_copy(vt_hbm, vt_v, sem_in),
            pltpu.async_copy(fuv_hbm, fuv_v, sem_in),
            pltpu.async_copy(pf_hbm.at[pl.ds(wid * PW, PW)], pf_all, sem_in),
            pltpu.async_copy(bary_hbm.at[pl.ds(wid * 3 * PW, 3 * PW)],
                             bary_all.at[pl.ds(16, 3 * PW)], sem_in),
        ]
        for h_ in ins:
            h_.wait()

        def gather_copies(par, sem):
            return [
                pltpu.make_async_copy(
                    verts_hbm.at[idx_v.at[par * B + b]],
                    gbuf.at[pl.ds((par * B + b) * 3 * C, 3 * C)], sem)
                for b in range(B)
            ]

        def fire(c, par):
            sem = sem_g0 if par == 0 else sem_g1
            for g in range(C // _L):
                pfv = plsc.load_gather(pf_all, [splat(c * C + g * _L) + iota])
                pfv = jnp.minimum(jnp.maximum(pfv, 0), F - 1)
                for kk in range(3):
                    uv = plsc.load_gather(fuv_v, [pfv * 3 + kk])
                    vi = plsc.load_gather(vt_v, [uv])
                    for b in range(B):
                        plsc.store_scatter(
                            idx_v,
                            [splat(par * B + b), splat(kk * C + g * _L) + iota],
                            vi + b * V)
            for cpd in gather_copies(par, sem):
                cpd.start()

        def drain(par):
            sem = sem_g0 if par == 0 else sem_g1
            for cpd in gather_copies(par, sem):
                cpd.wait()

        def combine(c, par):
            @pl.loop(0, C)
            def _(p):
                wbase = 16 + (c * C + p) * 3
                w = [plsc.load_gather(bary_all, [splat(wbase + kk)])
                     for kk in range(3)]
                for b in range(B):
                    rbase = (par * B + b) * 3 * C
                    obase = (par * B + b) * C + p
                    for h in range(D // _L):
                        lane = iota + h * _L
                        r = [plsc.load_gather(gbuf, [splat(rbase + kk * C + p), lane])
                             for kk in range(3)]
                        acc = w[0] * r[0] + w[1] * r[1] + w[2] * r[2]
                        plsc.store_scatter(obuf, [splat(obase), lane], acc)

        def out_copies(c, par, sem):
            return [
                pltpu.make_async_copy(
                    obuf.at[pl.ds((par * B + b) * C, C)],
                    out_hbm.at[b, pl.ds(wid * PW + c * C, C)], sem)
                for b in range(B)
            ]

        fire(0, 0)

        @pl.loop(0, NCHUNK, step=2)
        def _(c0):
            fire(c0 + 1, 1)
            drain(0)

            @pl.when(c0 >= 2)
            def _():
                for cpd in out_copies(c0 - 2, 0, sem_o0):
                    cpd.wait()

            combine(c0, 0)
            for cpd in out_copies(c0, 0, sem_o0):
                cpd.start()

            @pl.when(c0 + 2 < NCHUNK)
            def _():
                fire(c0 + 2, 0)

            drain(1)

            @pl.when(c0 >= 2)
            def _():
                for cpd in out_copies(c0 - 1, 1, sem_o1):
                    cpd.wait()

            combine(c0 + 1, 1)
            for cpd in out_copies(c0 + 1, 1, sem_o1):
                cpd.start()

        for cpd in out_copies(NCHUNK - 2, 0, sem_o0):
            cpd.wait()
        for cpd in out_copies(NCHUNK - 1, 1, sem_o1):
            cpd.wait()

    return k(verts_flat, bary_flat, vt_to_v, fuv_flat, pf_flat)


def kernel(verts_attr, bary_coords, vt_to_v_index, faces_uv, pix_to_face):
    B, V, D = verts_attr.shape
    F = faces_uv.shape[0]
    H, W = pix_to_face.shape
    P = H * W
    vt = vt_to_v_index.astype(jnp.int32)
    vt_pad = (-vt.shape[0]) % 16
    if vt_pad:
        vt = jnp.pad(vt, (0, vt_pad))
    out = _uv_render_sc(
        verts_attr.reshape(B * V, D),
        bary_coords.astype(jnp.float32).reshape(P * 3),
        vt,
        faces_uv.astype(jnp.int32).reshape(F * 3),
        pix_to_face.astype(jnp.int32).reshape(P),
        B=B, V=V, D=D, F=F, P=P,
    )
    return out.reshape(B, H, W, D)
